# baseline (speedup 1.0000x reference)
import functools

import jax
import jax.numpy as jnp
from jax.experimental import pallas as pl
from jax.experimental.pallas import tpu as pltpu

EPS = 1e-5
_VMEM_LIMIT = 48 * 1024 * 1024
_PAD = 128


def _pool_kernel(x_ref, p_ref, o_ref, *, c, h, w):
    hm = jnp.maximum(x_ref[0, :, :, 0, :], x_ref[0, :, :, 1, :])
    x2 = hm.reshape(c * h, 2 * w)
    rolled = jnp.concatenate([x2[:, 1:], x2[:, :1]], axis=1)
    maxed = jnp.maximum(x2, rolled).astype(jnp.bfloat16)
    sel = jnp.dot(maxed, p_ref[...], preferred_element_type=jnp.float32)
    o_ref[0] = sel.astype(jnp.bfloat16).reshape(c, h, w)


def _im2col9(xflat, cin, hw, w):
    col = jax.lax.broadcasted_iota(jnp.int32, (cin, hw), 1) % w
    zero = jnp.zeros_like(xflat)
    x_l = jnp.where(col == w - 1, zero, xflat)
    x_r = jnp.where(col == 0, zero, xflat)
    bigs = [jnp.pad(v, ((0, 0), (_PAD, _PAD))) for v in (x_l, xflat, x_r)]
    pieces = []
    for ky in range(3):
        for kx in range(3):
            s = (ky - 1) * w + (kx - 1)
            pieces.append(
                jax.lax.slice(bigs[kx], (0, _PAD + s), (cin, _PAD + s + hw)))
    return jnp.concatenate(pieces, axis=0)


def _store_stats(st_ref, acc):
    st_ref[0, :, 0:1] = jnp.sum(acc, axis=1, keepdims=True)
    st_ref[0, :, 1:2] = jnp.sum(acc * acc, axis=1, keepdims=True)


def _bn_scale_shift(st_ref, g_ref, be_ref, cnt):
    s = jnp.sum(st_ref[...], axis=0)
    mu = s[:, 0:1] * (1.0 / cnt)
    var = s[:, 1:2] * (1.0 / cnt) - mu * mu
    scale = g_ref[...] * jax.lax.rsqrt(var + EPS)
    shift = be_ref[...] - mu * scale
    return scale, shift


def _conv1_stats_kernel(x_ref, w_ref, st_ref, *, cin, hw, w):
    rhs = _im2col9(x_ref[0], cin, hw, w)
    acc = jnp.dot(w_ref[...], rhs, preferred_element_type=jnp.float32)
    _store_stats(st_ref, acc)


def _main_kernel(x_ref, w1_ref, st1_ref, g1_ref, be1_ref, w2_ref,
                 y2_ref, st2_ref, *, cin, c1, hw, w, cnt):
    rhs = _im2col9(x_ref[0], cin, hw, w)
    acc = jnp.dot(w1_ref[...], rhs, preferred_element_type=jnp.float32)
    scale, shift = _bn_scale_shift(st1_ref, g1_ref, be1_ref, cnt)
    xc = jnp.maximum(acc * scale + shift, 0.0).astype(jnp.bfloat16)
    rhs2 = _im2col9(xc, c1, hw, w)
    acc2 = jnp.dot(w2_ref[...], rhs2, preferred_element_type=jnp.float32)
    _store_stats(st2_ref, acc2)
    y2_ref[0] = acc2.astype(jnp.bfloat16)


def _out_kernel(y2_ref, st2_ref, g2_ref, be2_ref, o_ref, *, cnt):
    scale, shift = _bn_scale_shift(st2_ref, g2_ref, be2_ref, cnt)
    o_ref[0] = jnp.maximum(y2_ref[0].astype(jnp.float32) * scale + shift, 0.0)


def kernel(x, w1, b1, g1, be1, w2, b2, g2, be2):
    del b1, b2
    n, cin, h2, w2s = x.shape
    h, w = h2 // 2, w2s // 2
    hw = h * w
    c1 = w1.shape[0]
    c2 = w2.shape[0]
    cnt = float(n * hw)
    par = pltpu.CompilerParams(dimension_semantics=("parallel",),
                               vmem_limit_bytes=_VMEM_LIMIT)

    xv = x.reshape(n, cin, h, 2, 2 * w)
    psel = (jax.lax.broadcasted_iota(jnp.int32, (2 * w, w), 0) ==
            2 * jax.lax.broadcasted_iota(jnp.int32, (2 * w, w), 1)
            ).astype(jnp.bfloat16)
    xpool = pl.pallas_call(
        functools.partial(_pool_kernel, c=cin, h=h, w=w),
        out_shape=jax.ShapeDtypeStruct((n, cin, h, w), jnp.bfloat16),
        grid=(n,),
        in_specs=[
            pl.BlockSpec((1, cin, h, 2, 2 * w), lambda i: (i, 0, 0, 0, 0)),
            pl.BlockSpec((2 * w, w), lambda i: (0, 0)),
        ],
        out_specs=pl.BlockSpec((1, cin, h, w), lambda i: (i, 0, 0, 0)),
        compiler_params=par,
    )(xv, psel)
    xflat = xpool.reshape(n, cin, hw)

    w1l = jnp.transpose(w1, (0, 2, 3, 1)).reshape(c1, 9 * cin).astype(jnp.bfloat16)
    w2l = jnp.transpose(w2, (0, 2, 3, 1)).reshape(c2, 9 * c1).astype(jnp.bfloat16)
    g1c, be1c = g1.reshape(c1, 1), be1.reshape(c1, 1)
    g2c, be2c = g2.reshape(c2, 1), be2.reshape(c2, 1)

    st1 = pl.pallas_call(
        functools.partial(_conv1_stats_kernel, cin=cin, hw=hw, w=w),
        out_shape=jax.ShapeDtypeStruct((n, c1, 2), jnp.float32),
        grid=(n,),
        in_specs=[
            pl.BlockSpec((1, cin, hw), lambda i: (i, 0, 0)),
            pl.BlockSpec((c1, 9 * cin), lambda i: (0, 0)),
        ],
        out_specs=pl.BlockSpec((1, c1, 2), lambda i: (i, 0, 0)),
        compiler_params=par,
    )(xflat, w1l)

    y2, st2 = pl.pallas_call(
        functools.partial(_main_kernel, cin=cin, c1=c1, hw=hw, w=w, cnt=cnt),
        out_shape=(jax.ShapeDtypeStruct((n, c2, hw), jnp.bfloat16),
                   jax.ShapeDtypeStruct((n, c2, 2), jnp.float32)),
        grid=(n,),
        in_specs=[
            pl.BlockSpec((1, cin, hw), lambda i: (i, 0, 0)),
            pl.BlockSpec((c1, 9 * cin), lambda i: (0, 0)),
            pl.BlockSpec((n, c1, 2), lambda i: (0, 0, 0)),
            pl.BlockSpec((c1, 1), lambda i: (0, 0)),
            pl.BlockSpec((c1, 1), lambda i: (0, 0)),
            pl.BlockSpec((c2, 9 * c1), lambda i: (0, 0)),
        ],
        out_specs=(
            pl.BlockSpec((1, c2, hw), lambda i: (i, 0, 0)),
            pl.BlockSpec((1, c2, 2), lambda i: (i, 0, 0)),
        ),
        compiler_params=par,
    )(xflat, w1l, st1, g1c, be1c, w2l)

    out = pl.pallas_call(
        functools.partial(_out_kernel, cnt=cnt),
        out_shape=jax.ShapeDtypeStruct((n, c2, hw), jnp.float32),
        grid=(n,),
        in_specs=[
            pl.BlockSpec((1, c2, hw), lambda i: (i, 0, 0)),
            pl.BlockSpec((n, c2, 2), lambda i: (0, 0, 0)),
            pl.BlockSpec((c2, 1), lambda i: (0, 0)),
            pl.BlockSpec((c2, 1), lambda i: (0, 0)),
        ],
        out_specs=pl.BlockSpec((1, c2, hw), lambda i: (i, 0, 0)),
        compiler_params=par,
    )(y2, st2, g2c, be2c)

    return out.reshape(n, c2, h, w)


# speedup vs baseline: 1.0639x; 1.0639x over previous
import functools

import jax
import jax.numpy as jnp
from jax.experimental import pallas as pl
from jax.experimental.pallas import tpu as pltpu

EPS = 1e-5
_VMEM_LIMIT = 48 * 1024 * 1024
_PAD = 128


def _pool_kernel(x_ref, p_ref, o_ref, *, c, h, w):
    hm = jnp.maximum(x_ref[0, :, :, 0, :], x_ref[0, :, :, 1, :])
    x2 = hm.reshape(c * h, 2 * w)
    rolled = jnp.concatenate([x2[:, 1:], x2[:, :1]], axis=1)
    maxed = jnp.maximum(x2, rolled).astype(jnp.bfloat16)
    sel = jnp.dot(maxed, p_ref[...], preferred_element_type=jnp.float32)
    o_ref[0] = sel.astype(jnp.bfloat16).reshape(c, h, w)


def _im2col9(xflat, cin, hw, w):
    col = jax.lax.broadcasted_iota(jnp.int32, (cin, hw), 1) % w
    zero = jnp.zeros_like(xflat)
    x_l = jnp.where(col == w - 1, zero, xflat)
    x_r = jnp.where(col == 0, zero, xflat)
    bigs = [jnp.pad(v, ((0, 0), (_PAD, _PAD))) for v in (x_l, xflat, x_r)]
    pieces = []
    for ky in range(3):
        for kx in range(3):
            s = (ky - 1) * w + (kx - 1)
            pieces.append(
                jax.lax.slice(bigs[kx], (0, _PAD + s), (cin, _PAD + s + hw)))
    return jnp.concatenate(pieces, axis=0)


def _store_stats(st_ref, j, acc):
    st_ref[j, :, 0:1] = jnp.sum(acc, axis=1, keepdims=True)
    st_ref[j, :, 1:2] = jnp.sum(acc * acc, axis=1, keepdims=True)


def _bn_scale_shift(st_ref, g_ref, be_ref, cnt):
    s = jnp.sum(st_ref[...], axis=0)
    mu = s[:, 0:1] * (1.0 / cnt)
    var = s[:, 1:2] * (1.0 / cnt) - mu * mu
    scale = g_ref[...] * jax.lax.rsqrt(var + EPS)
    shift = be_ref[...] - mu * scale
    return scale, shift


def _conv1_kernel(x_ref, w_ref, y_ref, st_ref, *, pb, cin, hw, w):
    for j in range(pb):
        rhs = _im2col9(x_ref[j], cin, hw, w)
        acc = jnp.dot(w_ref[...], rhs, preferred_element_type=jnp.float32)
        _store_stats(st_ref, j, acc)
        y_ref[j] = acc.astype(jnp.bfloat16)


def _affine_conv2_kernel(y1_ref, st1_ref, g_ref, be_ref, w_ref,
                         y_ref, st_ref, *, pb, cin, hw, w, cnt):
    scale, shift = _bn_scale_shift(st1_ref, g_ref, be_ref, cnt)
    for j in range(pb):
        y1 = y1_ref[j].astype(jnp.float32)
        xc = jnp.maximum(y1 * scale + shift, 0.0).astype(jnp.bfloat16)
        rhs = _im2col9(xc, cin, hw, w)
        acc = jnp.dot(w_ref[...], rhs, preferred_element_type=jnp.float32)
        _store_stats(st_ref, j, acc)
        y_ref[j] = acc.astype(jnp.bfloat16)


def _affine_out_kernel(y2_ref, st2_ref, g_ref, be_ref, o_ref, *, pb, cnt):
    scale, shift = _bn_scale_shift(st2_ref, g_ref, be_ref, cnt)
    for j in range(pb):
        y2 = y2_ref[j].astype(jnp.float32)
        o_ref[j] = jnp.maximum(y2 * scale + shift, 0.0)


def kernel(x, w1, b1, g1, be1, w2, b2, g2, be2):
    del b1, b2
    n, cin, h2, w2s = x.shape
    h, w = h2 // 2, w2s // 2
    hw = h * w
    c1 = w1.shape[0]
    c2 = w2.shape[0]
    cnt = float(n * hw)
    pb = n // 2
    par = pltpu.CompilerParams(dimension_semantics=("parallel",),
                               vmem_limit_bytes=_VMEM_LIMIT)

    xv = x.reshape(n, cin, h, 2, 2 * w)
    psel = (jax.lax.broadcasted_iota(jnp.int32, (2 * w, w), 0) ==
            2 * jax.lax.broadcasted_iota(jnp.int32, (2 * w, w), 1)
            ).astype(jnp.bfloat16)
    xpool = pl.pallas_call(
        functools.partial(_pool_kernel, c=cin, h=h, w=w),
        out_shape=jax.ShapeDtypeStruct((n, cin, h, w), jnp.bfloat16),
        grid=(n,),
        in_specs=[
            pl.BlockSpec((1, cin, h, 2, 2 * w), lambda i: (i, 0, 0, 0, 0)),
            pl.BlockSpec((2 * w, w), lambda i: (0, 0)),
        ],
        out_specs=pl.BlockSpec((1, cin, h, w), lambda i: (i, 0, 0, 0)),
        compiler_params=par,
    )(xv, psel)
    xflat = xpool.reshape(n, cin, hw)

    w1l = jnp.transpose(w1, (0, 2, 3, 1)).reshape(c1, 9 * cin).astype(jnp.bfloat16)
    w2l = jnp.transpose(w2, (0, 2, 3, 1)).reshape(c2, 9 * c1).astype(jnp.bfloat16)
    g1c, be1c = g1.reshape(c1, 1), be1.reshape(c1, 1)
    g2c, be2c = g2.reshape(c2, 1), be2.reshape(c2, 1)

    y1, st1 = pl.pallas_call(
        functools.partial(_conv1_kernel, pb=pb, cin=cin, hw=hw, w=w),
        out_shape=(jax.ShapeDtypeStruct((n, c1, hw), jnp.bfloat16),
                   jax.ShapeDtypeStruct((n, c1, 2), jnp.float32)),
        grid=(n // pb,),
        in_specs=[
            pl.BlockSpec((pb, cin, hw), lambda i: (i, 0, 0)),
            pl.BlockSpec((c1, 9 * cin), lambda i: (0, 0)),
        ],
        out_specs=(
            pl.BlockSpec((pb, c1, hw), lambda i: (i, 0, 0)),
            pl.BlockSpec((pb, c1, 2), lambda i: (i, 0, 0)),
        ),
        compiler_params=par,
    )(xflat, w1l)

    y2, st2 = pl.pallas_call(
        functools.partial(_affine_conv2_kernel, pb=pb, cin=c1, hw=hw, w=w,
                          cnt=cnt),
        out_shape=(jax.ShapeDtypeStruct((n, c2, hw), jnp.bfloat16),
                   jax.ShapeDtypeStruct((n, c2, 2), jnp.float32)),
        grid=(n // pb,),
        in_specs=[
            pl.BlockSpec((pb, c1, hw), lambda i: (i, 0, 0)),
            pl.BlockSpec((n, c1, 2), lambda i: (0, 0, 0)),
            pl.BlockSpec((c1, 1), lambda i: (0, 0)),
            pl.BlockSpec((c1, 1), lambda i: (0, 0)),
            pl.BlockSpec((c2, 9 * c1), lambda i: (0, 0)),
        ],
        out_specs=(
            pl.BlockSpec((pb, c2, hw), lambda i: (i, 0, 0)),
            pl.BlockSpec((pb, c2, 2), lambda i: (i, 0, 0)),
        ),
        compiler_params=par,
    )(y1, st1, g1c, be1c, w2l)

    out = pl.pallas_call(
        functools.partial(_affine_out_kernel, pb=pb, cnt=cnt),
        out_shape=jax.ShapeDtypeStruct((n, c2, hw), jnp.float32),
        grid=(n // pb,),
        in_specs=[
            pl.BlockSpec((pb, c2, hw), lambda i: (i, 0, 0)),
            pl.BlockSpec((n, c2, 2), lambda i: (0, 0, 0)),
            pl.BlockSpec((c2, 1), lambda i: (0, 0)),
            pl.BlockSpec((c2, 1), lambda i: (0, 0)),
        ],
        out_specs=pl.BlockSpec((pb, c2, hw), lambda i: (i, 0, 0)),
        compiler_params=par,
    )(y2, st2, g2c, be2c)

    return out.reshape(n, c2, h, w)


# speedup vs baseline: 1.0935x; 1.0279x over previous
import functools

import jax
import jax.numpy as jnp
from jax.experimental import pallas as pl
from jax.experimental.pallas import tpu as pltpu

EPS = 1e-5
_VMEM_LIMIT = 48 * 1024 * 1024
_PAD = 128


def _im2col9(xflat, cin, hw, w):
    col = jax.lax.broadcasted_iota(jnp.int32, (cin, hw), 1) % w
    zero = jnp.zeros_like(xflat)
    x_l = jnp.where(col == w - 1, zero, xflat)
    x_r = jnp.where(col == 0, zero, xflat)
    bigs = [jnp.pad(v, ((0, 0), (_PAD, _PAD))) for v in (x_l, xflat, x_r)]
    pieces = []
    for ky in range(3):
        for kx in range(3):
            s = (ky - 1) * w + (kx - 1)
            pieces.append(
                jax.lax.slice(bigs[kx], (0, _PAD + s), (cin, _PAD + s + hw)))
    return jnp.concatenate(pieces, axis=0)


def _pool_kernel(x_ref, p_ref, o_ref, *, c, h, w):
    a = x_ref[0].reshape(c * h, 2, 2 * w)
    hm = jnp.maximum(a[:, 0, :], a[:, 1, :])
    rolled = jnp.concatenate([hm[:, 1:], hm[:, :1]], axis=1)
    maxed = jnp.maximum(hm, rolled).astype(jnp.bfloat16)
    sel = jnp.dot(maxed, p_ref[...], preferred_element_type=jnp.float32)
    o_ref[0] = sel.astype(jnp.bfloat16).reshape(c, h, w)


def _store_stats(st_ref, acc):
    st_ref[0, :, 0:1] = jnp.sum(acc, axis=1, keepdims=True)
    st_ref[0, :, 1:2] = jnp.sum(acc * acc, axis=1, keepdims=True)


def _conv1_kernel(x_ref, w_ref, y_ref, st_ref, *, cin, hw, w):
    rhs = _im2col9(x_ref[0], cin, hw, w)
    acc = jnp.dot(w_ref[...], rhs, preferred_element_type=jnp.float32)
    _store_stats(st_ref, acc)
    y_ref[0] = acc.astype(jnp.bfloat16)


def _affine_conv2_kernel(y1_ref, s_ref, b_ref, w_ref, y_ref, st_ref, *,
                         cin, hw, w):
    y1 = y1_ref[0].astype(jnp.float32)
    xc = jnp.maximum(y1 * s_ref[...] + b_ref[...], 0.0).astype(jnp.bfloat16)
    rhs = _im2col9(xc, cin, hw, w)
    acc = jnp.dot(w_ref[...], rhs, preferred_element_type=jnp.float32)
    _store_stats(st_ref, acc)
    y_ref[0] = acc.astype(jnp.bfloat16)


def _affine_out_kernel(y2_ref, s_ref, b_ref, o_ref):
    y2 = y2_ref[0].astype(jnp.float32)
    o_ref[0] = jnp.maximum(y2 * s_ref[...] + b_ref[...], 0.0)


def _finalize_bn(stats, gamma, beta, cnt):
    s = jnp.sum(stats[:, :, 0], axis=0)
    ss = jnp.sum(stats[:, :, 1], axis=0)
    mu = s / cnt
    var = ss / cnt - mu * mu
    scale = gamma * jax.lax.rsqrt(var + EPS)
    shift = beta - mu * scale
    return scale.reshape(-1, 1), shift.reshape(-1, 1)


def kernel(x, w1, b1, g1, be1, w2, b2, g2, be2):
    del b1, b2
    n, cin, h2, w2s = x.shape
    h, w = h2 // 2, w2s // 2
    hw = h * w
    c1 = w1.shape[0]
    c2 = w2.shape[0]

    xv = x.reshape(n, cin * h * 2, 2 * w)
    psel = (jax.lax.broadcasted_iota(jnp.int32, (2 * w, w), 0) ==
            2 * jax.lax.broadcasted_iota(jnp.int32, (2 * w, w), 1)
            ).astype(jnp.bfloat16)
    k_p = functools.partial(_pool_kernel, c=cin, h=h, w=w)
    xpool = pl.pallas_call(
        k_p,
        out_shape=jax.ShapeDtypeStruct((n, cin, h, w), jnp.bfloat16),
        grid=(n,),
        in_specs=[
            pl.BlockSpec((1, cin * h * 2, 2 * w), lambda i: (i, 0, 0)),
            pl.BlockSpec((2 * w, w), lambda i: (0, 0)),
        ],
        out_specs=pl.BlockSpec((1, cin, h, w), lambda i: (i, 0, 0, 0)),
        compiler_params=pltpu.CompilerParams(
            dimension_semantics=("parallel",),
            vmem_limit_bytes=_VMEM_LIMIT),
    )(xv, psel)
    xflat = xpool.reshape(n, cin, hw)

    w1l = jnp.transpose(w1, (0, 2, 3, 1)).reshape(c1, 9 * cin).astype(jnp.bfloat16)
    w2l = jnp.transpose(w2, (0, 2, 3, 1)).reshape(c2, 9 * c1).astype(jnp.bfloat16)

    k_a = functools.partial(_conv1_kernel, cin=cin, hw=hw, w=w)
    y1, st1 = pl.pallas_call(
        k_a,
        out_shape=(jax.ShapeDtypeStruct((n, c1, hw), jnp.bfloat16),
                   jax.ShapeDtypeStruct((n, c1, 2), jnp.float32)),
        grid=(n,),
        in_specs=[
            pl.BlockSpec((1, cin, hw), lambda i: (i, 0, 0)),
            pl.BlockSpec((c1, 9 * cin), lambda i: (0, 0)),
        ],
        out_specs=(
            pl.BlockSpec((1, c1, hw), lambda i: (i, 0, 0)),
            pl.BlockSpec((1, c1, 2), lambda i: (i, 0, 0)),
        ),
        compiler_params=pltpu.CompilerParams(
            dimension_semantics=("parallel",),
            vmem_limit_bytes=_VMEM_LIMIT),
    )(xflat, w1l)

    scale1, shift1 = _finalize_bn(st1, g1, be1, float(n * hw))

    k_b = functools.partial(_affine_conv2_kernel, cin=c1, hw=hw, w=w)
    y2, st2 = pl.pallas_call(
        k_b,
        out_shape=(jax.ShapeDtypeStruct((n, c2, hw), jnp.bfloat16),
                   jax.ShapeDtypeStruct((n, c2, 2), jnp.float32)),
        grid=(n,),
        in_specs=[
            pl.BlockSpec((1, c1, hw), lambda i: (i, 0, 0)),
            pl.BlockSpec((c1, 1), lambda i: (0, 0)),
            pl.BlockSpec((c1, 1), lambda i: (0, 0)),
            pl.BlockSpec((c2, 9 * c1), lambda i: (0, 0)),
        ],
        out_specs=(
            pl.BlockSpec((1, c2, hw), lambda i: (i, 0, 0)),
            pl.BlockSpec((1, c2, 2), lambda i: (i, 0, 0)),
        ),
        compiler_params=pltpu.CompilerParams(
            dimension_semantics=("parallel",),
            vmem_limit_bytes=_VMEM_LIMIT),
    )(y1, scale1, shift1, w2l)

    scale2, shift2 = _finalize_bn(st2, g2, be2, float(n * hw))

    out = pl.pallas_call(
        _affine_out_kernel,
        out_shape=jax.ShapeDtypeStruct((n, c2, hw), jnp.float32),
        grid=(n,),
        in_specs=[
            pl.BlockSpec((1, c2, hw), lambda i: (i, 0, 0)),
            pl.BlockSpec((c2, 1), lambda i: (0, 0)),
            pl.BlockSpec((c2, 1), lambda i: (0, 0)),
        ],
        out_specs=pl.BlockSpec((1, c2, hw), lambda i: (i, 0, 0)),
        compiler_params=pltpu.CompilerParams(
            dimension_semantics=("parallel",),
            vmem_limit_bytes=_VMEM_LIMIT),
    )(y2, scale2, shift2)

    return out.reshape(n, c2, h, w)


# speedup vs baseline: 1.1321x; 1.0352x over previous
import functools

import jax
import jax.numpy as jnp
from jax.experimental import pallas as pl
from jax.experimental.pallas import tpu as pltpu

EPS = 1e-5
_VMEM_LIMIT = 48 * 1024 * 1024
_PAD = 128


def _im2col9(xflat, cin, hw, w):
    col = jax.lax.broadcasted_iota(jnp.int32, (cin, hw), 1) % w
    zero = jnp.zeros_like(xflat)
    x_l = jnp.where(col == w - 1, zero, xflat)
    x_r = jnp.where(col == 0, zero, xflat)
    bigs = [jnp.pad(v, ((0, 0), (_PAD, _PAD))) for v in (x_l, xflat, x_r)]
    pieces = []
    for ky in range(3):
        for kx in range(3):
            s = (ky - 1) * w + (kx - 1)
            pieces.append(
                jax.lax.slice(bigs[kx], (0, _PAD + s), (cin, _PAD + s + hw)))
    return jnp.concatenate(pieces, axis=0)


def _pool_kernel(x_ref, p_ref, o_ref, *, c, h, w):
    hm = jnp.maximum(x_ref[0, :, :, 0, :], x_ref[0, :, :, 1, :])
    x2 = hm.reshape(c * h, 2 * w)
    rolled = jnp.concatenate([x2[:, 1:], x2[:, :1]], axis=1)
    maxed = jnp.maximum(x2, rolled).astype(jnp.bfloat16)
    sel = jnp.dot(maxed, p_ref[...], preferred_element_type=jnp.float32)
    o_ref[0] = sel.astype(jnp.bfloat16).reshape(c, h, w)


def _store_stats(st_ref, acc):
    st_ref[0, :, 0:1] = jnp.sum(acc, axis=1, keepdims=True)
    st_ref[0, :, 1:2] = jnp.sum(acc * acc, axis=1, keepdims=True)


def _bn_scale_shift(st_ref, g_ref, be_ref, cnt):
    s = jnp.sum(st_ref[...], axis=0)
    mu = s[:, 0:1] * (1.0 / cnt)
    var = s[:, 1:2] * (1.0 / cnt) - mu * mu
    scale = g_ref[...] * jax.lax.rsqrt(var + EPS)
    shift = be_ref[...] - mu * scale
    return scale, shift


def _conv1_kernel(x_ref, w_ref, y_ref, st_ref, *, cin, hw, w):
    rhs = _im2col9(x_ref[0], cin, hw, w)
    acc = jnp.dot(w_ref[...], rhs, preferred_element_type=jnp.float32)
    _store_stats(st_ref, acc)
    y_ref[0] = acc.astype(jnp.bfloat16)


def _affine_conv2_kernel(y1_ref, st1_ref, g_ref, be_ref, w_ref, y_ref,
                         st_ref, *, cin, hw, w, cnt):
    scale, shift = _bn_scale_shift(st1_ref, g_ref, be_ref, cnt)
    y1 = y1_ref[0].astype(jnp.float32)
    xc = jnp.maximum(y1 * scale + shift, 0.0).astype(jnp.bfloat16)
    rhs = _im2col9(xc, cin, hw, w)
    acc = jnp.dot(w_ref[...], rhs, preferred_element_type=jnp.float32)
    _store_stats(st_ref, acc)
    y_ref[0] = acc.astype(jnp.bfloat16)


def _affine_out_kernel(y2_ref, st2_ref, g_ref, be_ref, o_ref, *, cnt):
    scale, shift = _bn_scale_shift(st2_ref, g_ref, be_ref, cnt)
    y2 = y2_ref[0].astype(jnp.float32)
    o_ref[0] = jnp.maximum(y2 * scale + shift, 0.0)


def kernel(x, w1, b1, g1, be1, w2, b2, g2, be2):
    del b1, b2
    n, cin, h2, w2s = x.shape
    h, w = h2 // 2, w2s // 2
    hw = h * w
    c1 = w1.shape[0]
    c2 = w2.shape[0]

    xv = x.reshape(n, cin, h, 2, 2 * w)
    psel = (jax.lax.broadcasted_iota(jnp.int32, (2 * w, w), 0) ==
            2 * jax.lax.broadcasted_iota(jnp.int32, (2 * w, w), 1)
            ).astype(jnp.bfloat16)
    k_p = functools.partial(_pool_kernel, c=cin, h=h, w=w)
    xpool = pl.pallas_call(
        k_p,
        out_shape=jax.ShapeDtypeStruct((n, cin, h, w), jnp.bfloat16),
        grid=(n,),
        in_specs=[
            pl.BlockSpec((1, cin, h, 2, 2 * w), lambda i: (i, 0, 0, 0, 0)),
            pl.BlockSpec((2 * w, w), lambda i: (0, 0)),
        ],
        out_specs=pl.BlockSpec((1, cin, h, w), lambda i: (i, 0, 0, 0)),
        compiler_params=pltpu.CompilerParams(
            dimension_semantics=("parallel",),
            vmem_limit_bytes=_VMEM_LIMIT),
    )(xv, psel)
    xflat = xpool.reshape(n, cin, hw)

    w1l = jnp.transpose(w1, (0, 2, 3, 1)).reshape(c1, 9 * cin).astype(jnp.bfloat16)
    w2l = jnp.transpose(w2, (0, 2, 3, 1)).reshape(c2, 9 * c1).astype(jnp.bfloat16)
    g1c, be1c = g1.reshape(c1, 1), be1.reshape(c1, 1)
    g2c, be2c = g2.reshape(c2, 1), be2.reshape(c2, 1)
    cnt = float(n * hw)

    k_a = functools.partial(_conv1_kernel, cin=cin, hw=hw, w=w)
    y1, st1 = pl.pallas_call(
        k_a,
        out_shape=(jax.ShapeDtypeStruct((n, c1, hw), jnp.bfloat16),
                   jax.ShapeDtypeStruct((n, c1, 2), jnp.float32)),
        grid=(n,),
        in_specs=[
            pl.BlockSpec((1, cin, hw), lambda i: (i, 0, 0)),
            pl.BlockSpec((c1, 9 * cin), lambda i: (0, 0)),
        ],
        out_specs=(
            pl.BlockSpec((1, c1, hw), lambda i: (i, 0, 0)),
            pl.BlockSpec((1, c1, 2), lambda i: (i, 0, 0)),
        ),
        compiler_params=pltpu.CompilerParams(
            dimension_semantics=("parallel",),
            vmem_limit_bytes=_VMEM_LIMIT),
    )(xflat, w1l)

    k_b = functools.partial(_affine_conv2_kernel, cin=c1, hw=hw, w=w, cnt=cnt)
    y2, st2 = pl.pallas_call(
        k_b,
        out_shape=(jax.ShapeDtypeStruct((n, c2, hw), jnp.bfloat16),
                   jax.ShapeDtypeStruct((n, c2, 2), jnp.float32)),
        grid=(n,),
        in_specs=[
            pl.BlockSpec((1, c1, hw), lambda i: (i, 0, 0)),
            pl.BlockSpec((n, c1, 2), lambda i: (0, 0, 0)),
            pl.BlockSpec((c1, 1), lambda i: (0, 0)),
            pl.BlockSpec((c1, 1), lambda i: (0, 0)),
            pl.BlockSpec((c2, 9 * c1), lambda i: (0, 0)),
        ],
        out_specs=(
            pl.BlockSpec((1, c2, hw), lambda i: (i, 0, 0)),
            pl.BlockSpec((1, c2, 2), lambda i: (i, 0, 0)),
        ),
        compiler_params=pltpu.CompilerParams(
            dimension_semantics=("parallel",),
            vmem_limit_bytes=_VMEM_LIMIT),
    )(y1, st1, g1c, be1c, w2l)

    out = pl.pallas_call(
        functools.partial(_affine_out_kernel, cnt=cnt),
        out_shape=jax.ShapeDtypeStruct((n, c2, hw), jnp.float32),
        grid=(n,),
        in_specs=[
            pl.BlockSpec((1, c2, hw), lambda i: (i, 0, 0)),
            pl.BlockSpec((n, c2, 2), lambda i: (0, 0, 0)),
            pl.BlockSpec((c2, 1), lambda i: (0, 0)),
            pl.BlockSpec((c2, 1), lambda i: (0, 0)),
        ],
        out_specs=pl.BlockSpec((1, c2, hw), lambda i: (i, 0, 0)),
        compiler_params=pltpu.CompilerParams(
            dimension_semantics=("parallel",),
            vmem_limit_bytes=_VMEM_LIMIT),
    )(y2, st2, g2c, be2c)

    return out.reshape(n, c2, h, w)


# speedup vs baseline: 1.1645x; 1.0287x over previous
import functools

import jax
import jax.numpy as jnp
from jax.experimental import pallas as pl
from jax.experimental.pallas import tpu as pltpu

EPS = 1e-5
_VMEM_LIMIT = 48 * 1024 * 1024
_PAD = 128


def _im2col9(xflat, cin, hw, w):
    col = jax.lax.broadcasted_iota(jnp.int32, (cin, hw), 1) % w
    zero = jnp.zeros_like(xflat)
    x_l = jnp.where(col == w - 1, zero, xflat)
    x_r = jnp.where(col == 0, zero, xflat)
    bigs = [jnp.pad(v, ((0, 0), (_PAD, _PAD))) for v in (x_l, xflat, x_r)]
    pieces = []
    for ky in range(3):
        for kx in range(3):
            s = (ky - 1) * w + (kx - 1)
            pieces.append(
                jax.lax.slice(bigs[kx], (0, _PAD + s), (cin, _PAD + s + hw)))
    return jnp.concatenate(pieces, axis=0)


def _pool_kernel(x_ref, o_ref, *, c, h, w):
    psel = (jax.lax.broadcasted_iota(jnp.int32, (2 * w, w), 0) ==
            2 * jax.lax.broadcasted_iota(jnp.int32, (2 * w, w), 1)
            ).astype(jnp.bfloat16)
    hm = jnp.maximum(x_ref[0, :, :, 0, :], x_ref[0, :, :, 1, :])
    x2 = hm.reshape(c * h, 2 * w)
    rolled = jnp.concatenate([x2[:, 1:], x2[:, :1]], axis=1)
    maxed = jnp.maximum(x2, rolled).astype(jnp.bfloat16)
    sel = jnp.dot(maxed, psel, preferred_element_type=jnp.float32)
    o_ref[0] = sel.astype(jnp.bfloat16).reshape(c, h, w)


def _store_stats(st_ref, acc):
    st_ref[0, :, 0:1] = jnp.sum(acc, axis=1, keepdims=True)
    st_ref[0, :, 1:2] = jnp.sum(acc * acc, axis=1, keepdims=True)


def _conv1_kernel(x_ref, w_ref, y_ref, st_ref, *, cin, hw, w):
    rhs = _im2col9(x_ref[0], cin, hw, w)
    acc = jnp.dot(w_ref[...], rhs, preferred_element_type=jnp.float32)
    _store_stats(st_ref, acc)
    y_ref[0] = acc.astype(jnp.bfloat16)


def _affine_conv2_kernel(y1_ref, s_ref, b_ref, w_ref, y_ref, st_ref, *,
                         cin, hw, w):
    y1 = y1_ref[0].astype(jnp.float32)
    xc = jnp.maximum(y1 * s_ref[...] + b_ref[...], 0.0).astype(jnp.bfloat16)
    rhs = _im2col9(xc, cin, hw, w)
    acc = jnp.dot(w_ref[...], rhs, preferred_element_type=jnp.float32)
    _store_stats(st_ref, acc)
    y_ref[0] = acc.astype(jnp.bfloat16)


def _affine_out_kernel(y2_ref, s_ref, b_ref, o_ref):
    y2 = y2_ref[0].astype(jnp.float32)
    o_ref[0] = jnp.maximum(y2 * s_ref[...] + b_ref[...], 0.0)


def _finalize_bn(stats, gamma, beta, cnt):
    s = jnp.sum(stats[:, :, 0], axis=0)
    ss = jnp.sum(stats[:, :, 1], axis=0)
    mu = s / cnt
    var = ss / cnt - mu * mu
    scale = gamma * jax.lax.rsqrt(var + EPS)
    shift = beta - mu * scale
    return scale.reshape(-1, 1), shift.reshape(-1, 1)


def kernel(x, w1, b1, g1, be1, w2, b2, g2, be2):
    del b1, b2
    n, cin, h2, w2s = x.shape
    h, w = h2 // 2, w2s // 2
    hw = h * w
    c1 = w1.shape[0]
    c2 = w2.shape[0]

    xv = x.reshape(n, cin, h, 2, 2 * w)
    k_p = functools.partial(_pool_kernel, c=cin, h=h, w=w)
    xpool = pl.pallas_call(
        k_p,
        out_shape=jax.ShapeDtypeStruct((n, cin, h, w), jnp.bfloat16),
        grid=(n,),
        in_specs=[
            pl.BlockSpec((1, cin, h, 2, 2 * w), lambda i: (i, 0, 0, 0, 0)),
        ],
        out_specs=pl.BlockSpec((1, cin, h, w), lambda i: (i, 0, 0, 0)),
        compiler_params=pltpu.CompilerParams(
            dimension_semantics=("parallel",),
            vmem_limit_bytes=_VMEM_LIMIT),
    )(xv)
    xflat = xpool.reshape(n, cin, hw)

    w1l = jnp.transpose(w1, (0, 2, 3, 1)).reshape(c1, 9 * cin).astype(jnp.bfloat16)
    w2l = jnp.transpose(w2, (0, 2, 3, 1)).reshape(c2, 9 * c1).astype(jnp.bfloat16)

    k_a = functools.partial(_conv1_kernel, cin=cin, hw=hw, w=w)
    y1, st1 = pl.pallas_call(
        k_a,
        out_shape=(jax.ShapeDtypeStruct((n, c1, hw), jnp.bfloat16),
                   jax.ShapeDtypeStruct((n, c1, 2), jnp.float32)),
        grid=(n,),
        in_specs=[
            pl.BlockSpec((1, cin, hw), lambda i: (i, 0, 0)),
            pl.BlockSpec((c1, 9 * cin), lambda i: (0, 0)),
        ],
        out_specs=(
            pl.BlockSpec((1, c1, hw), lambda i: (i, 0, 0)),
            pl.BlockSpec((1, c1, 2), lambda i: (i, 0, 0)),
        ),
        compiler_params=pltpu.CompilerParams(
            dimension_semantics=("parallel",),
            vmem_limit_bytes=_VMEM_LIMIT),
    )(xflat, w1l)

    scale1, shift1 = _finalize_bn(st1, g1, be1, float(n * hw))

    k_b = functools.partial(_affine_conv2_kernel, cin=c1, hw=hw, w=w)
    y2, st2 = pl.pallas_call(
        k_b,
        out_shape=(jax.ShapeDtypeStruct((n, c2, hw), jnp.bfloat16),
                   jax.ShapeDtypeStruct((n, c2, 2), jnp.float32)),
        grid=(n,),
        in_specs=[
            pl.BlockSpec((1, c1, hw), lambda i: (i, 0, 0)),
            pl.BlockSpec((c1, 1), lambda i: (0, 0)),
            pl.BlockSpec((c1, 1), lambda i: (0, 0)),
            pl.BlockSpec((c2, 9 * c1), lambda i: (0, 0)),
        ],
        out_specs=(
            pl.BlockSpec((1, c2, hw), lambda i: (i, 0, 0)),
            pl.BlockSpec((1, c2, 2), lambda i: (i, 0, 0)),
        ),
        compiler_params=pltpu.CompilerParams(
            dimension_semantics=("parallel",),
            vmem_limit_bytes=_VMEM_LIMIT),
    )(y1, scale1, shift1, w2l)

    scale2, shift2 = _finalize_bn(st2, g2, be2, float(n * hw))

    out = pl.pallas_call(
        _affine_out_kernel,
        out_shape=jax.ShapeDtypeStruct((n, c2, hw), jnp.float32),
        grid=(n,),
        in_specs=[
            pl.BlockSpec((1, c2, hw), lambda i: (i, 0, 0)),
            pl.BlockSpec((c2, 1), lambda i: (0, 0)),
            pl.BlockSpec((c2, 1), lambda i: (0, 0)),
        ],
        out_specs=pl.BlockSpec((1, c2, hw), lambda i: (i, 0, 0)),
        compiler_params=pltpu.CompilerParams(
            dimension_semantics=("parallel",),
            vmem_limit_bytes=_VMEM_LIMIT),
    )(y2, scale2, shift2)

    return out.reshape(n, c2, h, w)
